# baseline (speedup 1.0000x reference)
import functools

import jax
import jax.numpy as jnp
from jax import lax
from jax.experimental import pallas as pl
from jax.experimental.pallas import tpu as pltpu
from jax.experimental.pallas import tpu_sc as plsc

N = 10000
NP = 10240
E = 160000
EP = 163840
DH = 512
NF = 4
FC = 128
NG = 128
BN = 1024
NBLK = NP // BN
NT = 16
EPT = EP // NT
BE = 128
NB_E = EPT // BE
RPT = NP // NT
ZR = 16

@functools.cache
def _mesh():
    return plsc.VectorSubcoreMesh(core_axis_name="c", subcore_axis_name="s")


def _fill(ref, n, value):
    def body(k, _):
        ref[pl.ds(k * 16, 16)] = jnp.full((16,), value, ref.dtype)
        return 0
    lax.fori_loop(0, n // 16, body, 0)


NIBD = 8


def _sc_deg_body(dst2_hbm, deg_hbm, acc, dstv, onesv, zv, dsems, s2sems):
    cid = lax.axis_index("c")
    sid = lax.axis_index("s")
    nbd = NB_E // 2

    _fill(onesv, BE, 1.0)
    _fill(zv, RPT, 0.0)
    pltpu.sync_copy(zv, acc.at[pl.ds(sid * RPT, RPT)])
    plsc.subcore_barrier()

    def load_idx(i, q):
        row = (cid * NT + sid) * nbd + i
        pltpu.async_copy(dst2_hbm.at[row], dstv[q], dsems[q])

    def wait_scat(s, q):
        pltpu.make_async_copy(onesv, acc.at[dstv[q]], s2sems[s]).wait()

    for q in range(NIBD):
        load_idx(q, q)

    def step(t, _):
        for u in range(NIBD):
            i = t * NIBD + u
            s = u % 2

            @pl.when(i >= 2)
            def _():
                wait_scat(s, (u - 2) % NIBD)

                @pl.when(i - 2 + NIBD < nbd)
                def _():
                    load_idx(i - 2 + NIBD, (u - 2) % NIBD)

            pltpu.make_async_copy(dst2_hbm.at[0], dstv[u], dsems[u]).wait()
            pltpu.async_copy(onesv, acc.at[dstv[u]], s2sems[s], add=True)
        return 0
    lax.fori_loop(0, nbd // NIBD, step, 0)
    for k in range(2):
        i = nbd - 2 + k
        wait_scat(i % 2, i % NIBD)
    plsc.subcore_barrier()
    pltpu.sync_copy(acc.at[pl.ds(sid * RPT, RPT)],
                    deg_hbm.at[cid, pl.ds(sid * RPT, RPT)])


@functools.cache
def _sc_deg():
    return pl.kernel(
        _sc_deg_body,
        out_type=jax.ShapeDtypeStruct((2, NP), jnp.float32),
        mesh=_mesh(),
        scratch_types=[
            pltpu.VMEM_SHARED((NP,), jnp.float32),
            [pltpu.VMEM((BE,), jnp.int32) for _ in range(NIBD)],
            pltpu.VMEM((BE,), jnp.float32),
            pltpu.VMEM((RPT,), jnp.float32),
            [pltpu.SemaphoreType.DMA for _ in range(NIBD)],
            [pltpu.SemaphoreType.DMA for _ in range(2)],
        ],
    )


NRB = 2
NIB = 8
KSC = 1


def _sc_scat_body(g_hbm, src2_hbm, dst2_hbm, out_hbm,
                  acc, srcv, dstv, rows, zvb, gsems, ssems, dsems, s2sems):
    cid = lax.axis_index("c")
    sid = lax.axis_index("s")

    def zero_acc():
        def zacc(k, _):
            pltpu.sync_copy(zvb, acc.at[pl.ds(sid * RPT + k * ZR, ZR)])
            return 0
        lax.fori_loop(0, RPT // ZR, zacc, 0)

    def zfill(k, _):
        zvb[k // 8, pl.ds((k % 8) * 16, 16)] = jnp.zeros((16,), jnp.float32)
        return 0
    lax.fori_loop(0, ZR * 8, zfill, 0)
    zero_acc()
    plsc.subcore_barrier()

    def load_idx(i, q):
        row = sid * NB_E + i
        pltpu.async_copy(src2_hbm.at[row], srcv[q], ssems[q])
        pltpu.async_copy(dst2_hbm.at[row], dstv[q], dsems[q])

    def wait_scat(b, q):
        pltpu.make_async_copy(rows[b], acc.at[dstv[q]], s2sems[b]).wait()

    def start_gather(i, q, b, off):
        pltpu.make_async_copy(src2_hbm.at[0], srcv[q], ssems[q]).wait()
        for k in range(BE // 16):
            srcv[q][pl.ds(k * 16, 16)] = srcv[q][pl.ds(k * 16, 16)] + off
        pltpu.async_copy(g_hbm.at[srcv[q]], rows[b], gsems[b])

    for j in range(NF // 2):
        fc = cid * (NF // 2) + j
        off = fc * NP

        for q in range(NIB):
            load_idx(q, q)
        for b in range(KSC):
            start_gather(b, b, b, off)

        def step(t, _):
            for u in range(NIB):
                i = t * NIB + u
                q = u
                b = u % NRB

                @pl.when(i >= KSC)
                def _():
                    wait_scat((u - KSC) % NRB, (u - KSC) % NIB)

                    @pl.when(i - KSC + NIB < NB_E)
                    def _():
                        load_idx(i - KSC + NIB, (u - KSC) % NIB)

                pltpu.make_async_copy(
                    g_hbm.at[srcv[q]], rows[b], gsems[b]).wait()
                pltpu.make_async_copy(dst2_hbm.at[0], dstv[q], dsems[q]).wait()
                pltpu.async_copy(rows[b], acc.at[dstv[q]], s2sems[b], add=True)

                @pl.when(i + KSC < NB_E)
                def _():
                    start_gather(i + KSC, (u + KSC) % NIB, (u + KSC) % NRB, off)
            return 0
        lax.fori_loop(0, NB_E // NIB, step, 0)

        for k in range(KSC):
            i = NB_E - KSC + k
            wait_scat(i % NRB, i % NIB)

        plsc.subcore_barrier()

        def wout(k, _):
            r0 = sid * RPT + k * ZR
            pltpu.sync_copy(acc.at[pl.ds(r0, ZR)], out_hbm.at[fc, pl.ds(r0, ZR)])
            return 0
        lax.fori_loop(0, RPT // ZR, wout, 0)
        if j + 1 < NF // 2:
            plsc.subcore_barrier()
            zero_acc()
            plsc.subcore_barrier()


@functools.cache
def _sc_scat():
    return pl.kernel(
        _sc_scat_body,
        out_type=jax.ShapeDtypeStruct((NF, NP, FC), jnp.float32),
        mesh=_mesh(),
        scratch_types=[
            pltpu.VMEM_SHARED((NP, FC), jnp.float32),
            [pltpu.VMEM((BE,), jnp.int32) for _ in range(NIB)],
            [pltpu.VMEM((BE,), jnp.int32) for _ in range(NIB)],
            [pltpu.VMEM((BE, FC), jnp.float32) for _ in range(NRB)],
            pltpu.VMEM((ZR, FC), jnp.float32),
            [pltpu.SemaphoreType.DMA for _ in range(NRB)],
            [pltpu.SemaphoreType.DMA for _ in range(NIB)],
            [pltpu.SemaphoreType.DMA for _ in range(NIB)],
            [pltpu.SemaphoreType.DMA for _ in range(NRB)],
        ],
    )
assert NB_E % NIB == 0 and NIB % NRB == 0


def _tc1_body(x_ref, w_ref, deg_ref, out_ref):
    dinv = lax.rsqrt(deg_ref[...] + 1.0)
    h = jnp.dot(x_ref[...], w_ref[...],
                preferred_element_type=jnp.float32)
    g = h * dinv
    for f in range(NF):
        out_ref[f] = g[:, f * FC:(f + 1) * FC]


def _tc1(xp, w1p, deg2):
    return pl.pallas_call(
        _tc1_body,
        grid=(NBLK,),
        in_specs=[
            pl.BlockSpec((BN, 16), lambda i: (i, 0)),
            pl.BlockSpec((16, DH), lambda i: (0, 0)),
            pl.BlockSpec((BN, 1), lambda i: (i, 0)),
        ],
        out_specs=pl.BlockSpec((NF, BN, FC), lambda i: (0, i, 0)),
        out_shape=jax.ShapeDtypeStruct((NF, NP, FC), jnp.float32),
    )(xp, w1p, deg2)


def _tc_mid_body(r_ref, g_ref, deg_ref, b_ref, w_ref, out_ref):
    dinv = lax.rsqrt(deg_ref[...] + 1.0)
    acc = jnp.zeros((BN, DH), jnp.float32)
    for f in range(NF):
        xf = jnp.maximum(
            (r_ref[f] + g_ref[f]) * dinv + b_ref[0:1, f * FC:(f + 1) * FC], 0.0)
        acc = acc + jnp.dot(xf, w_ref[f * FC:(f + 1) * FC, :],
                            preferred_element_type=jnp.float32)
    gn = acc * dinv
    for f in range(NF):
        out_ref[f] = gn[:, f * FC:(f + 1) * FC]


def _tc_mid(r, g, deg2, b, w):
    return pl.pallas_call(
        _tc_mid_body,
        grid=(NBLK,),
        in_specs=[
            pl.BlockSpec((NF, BN, FC), lambda i: (0, i, 0)),
            pl.BlockSpec((NF, BN, FC), lambda i: (0, i, 0)),
            pl.BlockSpec((BN, 1), lambda i: (i, 0)),
            pl.BlockSpec((1, DH), lambda i: (0, 0)),
            pl.BlockSpec((DH, DH), lambda i: (0, 0)),
        ],
        out_specs=pl.BlockSpec((NF, BN, FC), lambda i: (0, i, 0)),
        out_shape=jax.ShapeDtypeStruct((NF, NP, FC), jnp.float32),
    )(r, g, deg2, b, w)


def _tc4_body(r_ref, g_ref, deg_ref, b_ref, batch_ref, wl_ref, bl_ref,
              out_ref, sums_ref, cnt_ref):
    i = pl.program_id(0)

    @pl.when(i == 0)
    def _():
        sums_ref[...] = jnp.zeros((NG, DH), jnp.float32)
        cnt_ref[...] = jnp.zeros((NG, 1), jnp.float32)

    dinv = lax.rsqrt(deg_ref[...] + 1.0)
    bvec = batch_ref[...]
    gid = lax.broadcasted_iota(jnp.int32, (BN, NG), 1)
    P = jnp.where(bvec == gid, 1.0, 0.0)
    ones = jnp.ones((BN, 1), jnp.float32)
    cnt_ref[...] += lax.dot_general(
        P, ones, (((0,), (0,)), ((), ())),
        preferred_element_type=jnp.float32)
    for f in range(NF):
        h3f = (r_ref[f] + g_ref[f]) * dinv + b_ref[0:1, f * FC:(f + 1) * FC]
        sums_ref[:, f * FC:(f + 1) * FC] += lax.dot_general(
            P, h3f, (((0,), (0,)), ((), ())),
            preferred_element_type=jnp.float32)

    @pl.when(i == NBLK - 1)
    def _():
        pooled = sums_ref[...] / jnp.maximum(cnt_ref[...], 1.0)
        out_ref[...] = jnp.dot(pooled, wl_ref[...],
                               preferred_element_type=jnp.float32) + bl_ref[...]


def _tc4(r, g, deg2, b, batchp, wl, bl):
    return pl.pallas_call(
        _tc4_body,
        grid=(NBLK,),
        in_specs=[
            pl.BlockSpec((NF, BN, FC), lambda i: (0, i, 0)),
            pl.BlockSpec((NF, BN, FC), lambda i: (0, i, 0)),
            pl.BlockSpec((BN, 1), lambda i: (i, 0)),
            pl.BlockSpec((1, DH), lambda i: (0, 0)),
            pl.BlockSpec((BN, 1), lambda i: (i, 0)),
            pl.BlockSpec((DH, 1), lambda i: (0, 0)),
            pl.BlockSpec((1, 1), lambda i: (0, 0)),
        ],
        out_specs=pl.BlockSpec((NG, 1), lambda i: (0, 0)),
        out_shape=jax.ShapeDtypeStruct((NG, 1), jnp.float32),
        scratch_shapes=[
            pltpu.VMEM((NG, DH), jnp.float32),
            pltpu.VMEM((NG, 1), jnp.float32),
        ],
    )(r, g, deg2, b, batchp, wl, bl)


def kernel(x, edge_index, batch, W1, b1, W2, b2, W3, b3, Wlin, blin):
    xp = jnp.pad(x, ((0, NP - N), (0, 16 - x.shape[1])))
    w1p = jnp.pad(W1, ((0, 16 - W1.shape[0]), (0, 0)))
    padidx = N + (jnp.arange(EP - E, dtype=jnp.int32) % (NP - N))
    src = jnp.concatenate([edge_index[0], padidx]).reshape(EP // BE, BE)
    dst = jnp.concatenate([edge_index[1], padidx])
    dst2 = dst.reshape(EP // BE, BE)
    batchp = jnp.pad(batch, (0, NP - N), constant_values=NG).reshape(NP, 1)

    deg2 = _sc_deg()(dst2).sum(axis=0).reshape(NP, 1)
    g1 = _tc1(xp, w1p, deg2)
    r1 = _sc_scat()(g1.reshape(NF * NP, FC), src, dst2)
    g2 = _tc_mid(r1, g1, deg2, b1.reshape(1, DH), W2)
    r2 = _sc_scat()(g2.reshape(NF * NP, FC), src, dst2)
    g3 = _tc_mid(r2, g2, deg2, b2.reshape(1, DH), W3)
    r3 = _sc_scat()(g3.reshape(NF * NP, FC), src, dst2)
    return _tc4(r3, g3, deg2, b3.reshape(1, DH), batchp, Wlin,
                blin.reshape(1, 1))


# speedup vs baseline: 1.1862x; 1.1862x over previous
import functools

import jax
import jax.numpy as jnp
from jax import lax
from jax.experimental import pallas as pl
from jax.experimental.pallas import tpu as pltpu
from jax.experimental.pallas import tpu_sc as plsc

N = 10000
NP = 10240
E = 160000
EP = 163840
DH = 512
NF = 4
FC = 128
NG = 128
BN = 1024
NBLK = NP // BN
NT = 16
EPT = EP // NT
BE = 80
NB_E = EPT // BE
RPT = NP // NT
ZR = 32

@functools.cache
def _mesh():
    return plsc.VectorSubcoreMesh(core_axis_name="c", subcore_axis_name="s")


def _fill(ref, n, value):
    def body(k, _):
        ref[pl.ds(k * 16, 16)] = jnp.full((16,), value, ref.dtype)
        return 0
    lax.fori_loop(0, n // 16, body, 0)


NIBD = 8


def _sc_deg_body(dst2_hbm, deg_hbm, acc, dstv, onesv, zv, dsems, s2sems):
    cid = lax.axis_index("c")
    sid = lax.axis_index("s")
    nbd = NB_E // 2

    _fill(onesv, BE, 1.0)
    _fill(zv, RPT, 0.0)
    pltpu.sync_copy(zv, acc.at[pl.ds(sid * RPT, RPT)])
    plsc.subcore_barrier()

    def load_idx(i, q):
        row = (cid * NT + sid) * nbd + i
        pltpu.async_copy(dst2_hbm.at[row], dstv[q], dsems[q])

    def wait_scat(s, q):
        pltpu.make_async_copy(onesv, acc.at[dstv[q]], s2sems[s]).wait()

    for q in range(NIBD):
        load_idx(q, q)

    def step(t, _):
        for u in range(NIBD):
            i = t * NIBD + u
            s = u % 2

            @pl.when(i >= 2)
            def _():
                wait_scat(s, (u - 2) % NIBD)

                @pl.when(i - 2 + NIBD < nbd)
                def _():
                    load_idx(i - 2 + NIBD, (u - 2) % NIBD)

            pltpu.make_async_copy(dst2_hbm.at[0], dstv[u], dsems[u]).wait()
            pltpu.async_copy(onesv, acc.at[dstv[u]], s2sems[s], add=True)
        return 0
    lax.fori_loop(0, nbd // NIBD, step, 0)
    for k in range(2):
        i = nbd - 2 + k
        wait_scat(i % 2, i % NIBD)
    plsc.subcore_barrier()
    pltpu.sync_copy(acc.at[pl.ds(sid * RPT, RPT)],
                    deg_hbm.at[cid, pl.ds(sid * RPT, RPT)])


@functools.cache
def _sc_deg():
    return pl.kernel(
        _sc_deg_body,
        out_type=jax.ShapeDtypeStruct((2, NP), jnp.float32),
        mesh=_mesh(),
        scratch_types=[
            pltpu.VMEM_SHARED((NP,), jnp.float32),
            [pltpu.VMEM((BE,), jnp.int32) for _ in range(NIBD)],
            pltpu.VMEM((BE,), jnp.float32),
            pltpu.VMEM((RPT,), jnp.float32),
            [pltpu.SemaphoreType.DMA for _ in range(NIBD)],
            [pltpu.SemaphoreType.DMA for _ in range(2)],
        ],
    )


NRB = 4
NIB = 8
KSC = 2


def _sc_scat_body(g_hbm, src2_hbm, dst2_hbm, out_hbm,
                  acc, srcv, dstv, rows, zvb, gsems, ssems, dsems, s2sems):
    cid = lax.axis_index("c")
    sid = lax.axis_index("s")

    def zero_acc():
        def zacc(k, _):
            pltpu.sync_copy(zvb, acc.at[pl.ds(sid * RPT + k * ZR, ZR)])
            return 0
        lax.fori_loop(0, RPT // ZR, zacc, 0)

    def zfill(k, _):
        zvb[k // 8, pl.ds((k % 8) * 16, 16)] = jnp.zeros((16,), jnp.float32)
        return 0
    lax.fori_loop(0, ZR * 8, zfill, 0)
    zero_acc()
    plsc.subcore_barrier()

    def load_idx(i, q):
        row = sid * NB_E + i
        pltpu.async_copy(src2_hbm.at[row], srcv[q], ssems[q])
        pltpu.async_copy(dst2_hbm.at[row], dstv[q], dsems[q])

    def wait_scat(b, q):
        pltpu.make_async_copy(rows[b], acc.at[dstv[q]], s2sems[b]).wait()

    def start_gather(i, q, b, off):
        pltpu.make_async_copy(src2_hbm.at[0], srcv[q], ssems[q]).wait()
        for k in range(BE // 16):
            srcv[q][pl.ds(k * 16, 16)] = srcv[q][pl.ds(k * 16, 16)] + off
        pltpu.async_copy(g_hbm.at[srcv[q]], rows[b], gsems[b])

    for j in range(NF // 2):
        fc = cid * (NF // 2) + j
        off = fc * NP

        for q in range(NIB):
            load_idx(q, q)
        for b in range(KSC):
            start_gather(b, b, b, off)

        def step(t, _):
            for u in range(NIB):
                i = t * NIB + u
                q = u
                b = u % NRB

                @pl.when(i >= KSC)
                def _():
                    wait_scat((u - KSC) % NRB, (u - KSC) % NIB)

                    @pl.when(i - KSC + NIB < NB_E)
                    def _():
                        load_idx(i - KSC + NIB, (u - KSC) % NIB)

                pltpu.make_async_copy(
                    g_hbm.at[srcv[q]], rows[b], gsems[b]).wait()
                pltpu.make_async_copy(dst2_hbm.at[0], dstv[q], dsems[q]).wait()
                pltpu.async_copy(rows[b], acc.at[dstv[q]], s2sems[b], add=True)

                @pl.when(i + KSC < NB_E)
                def _():
                    start_gather(i + KSC, (u + KSC) % NIB, (u + KSC) % NRB, off)
            return 0
        lax.fori_loop(0, NB_E // NIB, step, 0)

        for k in range(KSC):
            i = NB_E - KSC + k
            wait_scat(i % NRB, i % NIB)

        plsc.subcore_barrier()

        def wout(k, _):
            r0 = sid * RPT + k * ZR
            pltpu.sync_copy(acc.at[pl.ds(r0, ZR)], out_hbm.at[fc, pl.ds(r0, ZR)])
            return 0
        lax.fori_loop(0, RPT // ZR, wout, 0)
        if j + 1 < NF // 2:
            plsc.subcore_barrier()
            zero_acc()
            plsc.subcore_barrier()


@functools.cache
def _sc_scat():
    return pl.kernel(
        _sc_scat_body,
        out_type=jax.ShapeDtypeStruct((NF, NP, FC), jnp.float32),
        mesh=_mesh(),
        scratch_types=[
            pltpu.VMEM_SHARED((NP, FC), jnp.float32),
            [pltpu.VMEM((BE,), jnp.int32) for _ in range(NIB)],
            [pltpu.VMEM((BE,), jnp.int32) for _ in range(NIB)],
            [pltpu.VMEM((BE, FC), jnp.float32) for _ in range(NRB)],
            pltpu.VMEM((ZR, FC), jnp.float32),
            [pltpu.SemaphoreType.DMA for _ in range(NRB)],
            [pltpu.SemaphoreType.DMA for _ in range(NIB)],
            [pltpu.SemaphoreType.DMA for _ in range(NIB)],
            [pltpu.SemaphoreType.DMA for _ in range(NRB)],
        ],
    )
assert NB_E % NIB == 0 and NIB % NRB == 0


def _tc1_body(x_ref, w_ref, deg_ref, out_ref):
    dinv = lax.rsqrt(deg_ref[...] + 1.0)
    h = jnp.dot(x_ref[...], w_ref[...],
                preferred_element_type=jnp.float32)
    g = h * dinv
    for f in range(NF):
        out_ref[f] = g[:, f * FC:(f + 1) * FC]


def _tc1(xp, w1p, deg2):
    return pl.pallas_call(
        _tc1_body,
        grid=(NBLK,),
        in_specs=[
            pl.BlockSpec((BN, 16), lambda i: (i, 0)),
            pl.BlockSpec((16, DH), lambda i: (0, 0)),
            pl.BlockSpec((BN, 1), lambda i: (i, 0)),
        ],
        out_specs=pl.BlockSpec((NF, BN, FC), lambda i: (0, i, 0)),
        out_shape=jax.ShapeDtypeStruct((NF, NP, FC), jnp.float32),
    )(xp, w1p, deg2)


def _tc_mid_body(r_ref, g_ref, deg_ref, b_ref, w_ref, out_ref):
    dinv = lax.rsqrt(deg_ref[...] + 1.0)
    acc = jnp.zeros((BN, DH), jnp.float32)
    for f in range(NF):
        xf = jnp.maximum(
            (r_ref[f] + g_ref[f]) * dinv + b_ref[0:1, f * FC:(f + 1) * FC], 0.0)
        acc = acc + jnp.dot(xf, w_ref[f * FC:(f + 1) * FC, :],
                            preferred_element_type=jnp.float32)
    gn = acc * dinv
    for f in range(NF):
        out_ref[f] = gn[:, f * FC:(f + 1) * FC]


def _tc_mid(r, g, deg2, b, w):
    return pl.pallas_call(
        _tc_mid_body,
        grid=(NBLK,),
        in_specs=[
            pl.BlockSpec((NF, BN, FC), lambda i: (0, i, 0)),
            pl.BlockSpec((NF, BN, FC), lambda i: (0, i, 0)),
            pl.BlockSpec((BN, 1), lambda i: (i, 0)),
            pl.BlockSpec((1, DH), lambda i: (0, 0)),
            pl.BlockSpec((DH, DH), lambda i: (0, 0)),
        ],
        out_specs=pl.BlockSpec((NF, BN, FC), lambda i: (0, i, 0)),
        out_shape=jax.ShapeDtypeStruct((NF, NP, FC), jnp.float32),
    )(r, g, deg2, b, w)


def _tc4_body(r_ref, g_ref, deg_ref, b_ref, batch_ref, wl_ref, bl_ref,
              out_ref, sums_ref, cnt_ref):
    i = pl.program_id(0)

    @pl.when(i == 0)
    def _():
        sums_ref[...] = jnp.zeros((NG, DH), jnp.float32)
        cnt_ref[...] = jnp.zeros((NG, 1), jnp.float32)

    dinv = lax.rsqrt(deg_ref[...] + 1.0)
    bvec = batch_ref[...]
    gid = lax.broadcasted_iota(jnp.int32, (BN, NG), 1)
    P = jnp.where(bvec == gid, 1.0, 0.0)
    ones = jnp.ones((BN, 1), jnp.float32)
    cnt_ref[...] += lax.dot_general(
        P, ones, (((0,), (0,)), ((), ())),
        preferred_element_type=jnp.float32)
    for f in range(NF):
        h3f = (r_ref[f] + g_ref[f]) * dinv + b_ref[0:1, f * FC:(f + 1) * FC]
        sums_ref[:, f * FC:(f + 1) * FC] += lax.dot_general(
            P, h3f, (((0,), (0,)), ((), ())),
            preferred_element_type=jnp.float32)

    @pl.when(i == NBLK - 1)
    def _():
        pooled = sums_ref[...] / jnp.maximum(cnt_ref[...], 1.0)
        out_ref[...] = jnp.dot(pooled, wl_ref[...],
                               preferred_element_type=jnp.float32) + bl_ref[...]


def _tc4(r, g, deg2, b, batchp, wl, bl):
    return pl.pallas_call(
        _tc4_body,
        grid=(NBLK,),
        in_specs=[
            pl.BlockSpec((NF, BN, FC), lambda i: (0, i, 0)),
            pl.BlockSpec((NF, BN, FC), lambda i: (0, i, 0)),
            pl.BlockSpec((BN, 1), lambda i: (i, 0)),
            pl.BlockSpec((1, DH), lambda i: (0, 0)),
            pl.BlockSpec((BN, 1), lambda i: (i, 0)),
            pl.BlockSpec((DH, 1), lambda i: (0, 0)),
            pl.BlockSpec((1, 1), lambda i: (0, 0)),
        ],
        out_specs=pl.BlockSpec((NG, 1), lambda i: (0, 0)),
        out_shape=jax.ShapeDtypeStruct((NG, 1), jnp.float32),
        scratch_shapes=[
            pltpu.VMEM((NG, DH), jnp.float32),
            pltpu.VMEM((NG, 1), jnp.float32),
        ],
    )(r, g, deg2, b, batchp, wl, bl)


def kernel(x, edge_index, batch, W1, b1, W2, b2, W3, b3, Wlin, blin):
    xp = jnp.pad(x, ((0, NP - N), (0, 16 - x.shape[1])))
    w1p = jnp.pad(W1, ((0, 16 - W1.shape[0]), (0, 0)))
    padidx = N + (jnp.arange(EP - E, dtype=jnp.int32) % (NP - N))
    src = jnp.concatenate([edge_index[0], padidx]).reshape(EP // BE, BE)
    dst = jnp.concatenate([edge_index[1], padidx])
    dst2 = dst.reshape(EP // BE, BE)
    batchp = jnp.pad(batch, (0, NP - N), constant_values=NG).reshape(NP, 1)

    deg2 = _sc_deg()(dst2).sum(axis=0).reshape(NP, 1)
    g1 = _tc1(xp, w1p, deg2)
    r1 = _sc_scat()(g1.reshape(NF * NP, FC), src, dst2)
    g2 = _tc_mid(r1, g1, deg2, b1.reshape(1, DH), W2)
    r2 = _sc_scat()(g2.reshape(NF * NP, FC), src, dst2)
    g3 = _tc_mid(r2, g2, deg2, b2.reshape(1, DH), W3)
    r3 = _sc_scat()(g3.reshape(NF * NP, FC), src, dst2)
    return _tc4(r3, g3, deg2, b3.reshape(1, DH), batchp, Wlin,
                blin.reshape(1, 1))


# speedup vs baseline: 1.2364x; 1.0423x over previous
import functools

import jax
import jax.numpy as jnp
from jax import lax
from jax.experimental import pallas as pl
from jax.experimental.pallas import tpu as pltpu
from jax.experimental.pallas import tpu_sc as plsc

N = 10000
NP = 10240
E = 160000
EP = 163840
DH = 512
NF = 4
FC = 128
NG = 128
BN = 1024
NBLK = NP // BN
NT = 16
EPT = EP // NT
BE = 80
NB_E = EPT // BE
RPT = NP // NT
ZR = 64

@functools.cache
def _mesh():
    return plsc.VectorSubcoreMesh(core_axis_name="c", subcore_axis_name="s")


def _fill(ref, n, value):
    def body(k, _):
        ref[pl.ds(k * 16, 16)] = jnp.full((16,), value, ref.dtype)
        return 0
    lax.fori_loop(0, n // 16, body, 0)


NIBD = 8


def _sc_deg_body(dst2_hbm, deg_hbm, acc, dstv, onesv, zv, dsems, s2sems):
    cid = lax.axis_index("c")
    sid = lax.axis_index("s")
    nbd = NB_E // 2

    _fill(onesv, BE, 1.0)
    _fill(zv, RPT, 0.0)
    pltpu.sync_copy(zv, acc.at[pl.ds(sid * RPT, RPT)])
    plsc.subcore_barrier()

    def load_idx(i, q):
        row = (cid * NT + sid) * nbd + i
        pltpu.async_copy(dst2_hbm.at[row], dstv[q], dsems[q])

    def wait_scat(s, q):
        pltpu.make_async_copy(onesv, acc.at[dstv[q]], s2sems[s]).wait()

    for q in range(NIBD):
        load_idx(q, q)

    def step(t, _):
        for u in range(NIBD):
            i = t * NIBD + u
            s = u % 2

            @pl.when(i >= 2)
            def _():
                wait_scat(s, (u - 2) % NIBD)

                @pl.when(i - 2 + NIBD < nbd)
                def _():
                    load_idx(i - 2 + NIBD, (u - 2) % NIBD)

            pltpu.make_async_copy(dst2_hbm.at[0], dstv[u], dsems[u]).wait()
            pltpu.async_copy(onesv, acc.at[dstv[u]], s2sems[s], add=True)
        return 0
    lax.fori_loop(0, nbd // NIBD, step, 0)
    for k in range(2):
        i = nbd - 2 + k
        wait_scat(i % 2, i % NIBD)
    plsc.subcore_barrier()
    pltpu.sync_copy(acc.at[pl.ds(sid * RPT, RPT)],
                    deg_hbm.at[cid, pl.ds(sid * RPT, RPT)])


@functools.cache
def _sc_deg():
    return pl.kernel(
        _sc_deg_body,
        out_type=jax.ShapeDtypeStruct((2, NP), jnp.float32),
        mesh=_mesh(),
        scratch_types=[
            pltpu.VMEM_SHARED((NP,), jnp.float32),
            [pltpu.VMEM((BE,), jnp.int32) for _ in range(NIBD)],
            pltpu.VMEM((BE,), jnp.float32),
            pltpu.VMEM((RPT,), jnp.float32),
            [pltpu.SemaphoreType.DMA for _ in range(NIBD)],
            [pltpu.SemaphoreType.DMA for _ in range(2)],
        ],
    )


NRB = 4
NIB = 8
KSC = 2


def _sc_scat_body(g_hbm, src2_hbm, dst2_hbm, out_hbm,
                  acc, srcv, dstv, rows, gsems, ssems, dsems, s2sems, wsem):
    cid = lax.axis_index("c")
    sid = lax.axis_index("s")

    def zero_acc():
        def zfill(k, _):
            rows[0][k // 8, pl.ds((k % 8) * 16, 16)] = jnp.zeros(
                (16,), jnp.float32)
            return 0
        lax.fori_loop(0, ZR * 8, zfill, 0)
        zsrc = rows[0].at[pl.ds(0, ZR)]

        def zacc(k, _):
            pltpu.async_copy(zsrc, acc.at[pl.ds(sid * RPT + k * ZR, ZR)], wsem)
            return 0
        lax.fori_loop(0, RPT // ZR, zacc, 0)

        def zdrain(k, _):
            pltpu.make_async_copy(
                zsrc, acc.at[pl.ds(sid * RPT, ZR)], wsem).wait()
            return 0
        lax.fori_loop(0, RPT // ZR, zdrain, 0)

    zero_acc()
    plsc.subcore_barrier()

    def load_idx(i, q):
        row = sid * NB_E + i
        pltpu.async_copy(src2_hbm.at[row], srcv[q], ssems[q])
        pltpu.async_copy(dst2_hbm.at[row], dstv[q], dsems[q])

    def wait_scat(b, q):
        pltpu.make_async_copy(rows[b], acc.at[dstv[q]], s2sems[b]).wait()

    def start_gather(i, q, b, off):
        pltpu.make_async_copy(src2_hbm.at[0], srcv[q], ssems[q]).wait()
        for k in range(BE // 16):
            srcv[q][pl.ds(k * 16, 16)] = srcv[q][pl.ds(k * 16, 16)] + off
        pltpu.async_copy(g_hbm.at[srcv[q]], rows[b], gsems[b])

    for j in range(NF // 2):
        fc = cid * (NF // 2) + j
        off = fc * NP

        for q in range(NIB):
            load_idx(q, q)
        for b in range(KSC):
            start_gather(b, b, b, off)

        def step(t, _):
            for u in range(NIB):
                i = t * NIB + u
                q = u
                b = u % NRB

                @pl.when(i >= KSC)
                def _():
                    wait_scat((u - KSC) % NRB, (u - KSC) % NIB)

                    @pl.when(i - KSC + NIB < NB_E)
                    def _():
                        load_idx(i - KSC + NIB, (u - KSC) % NIB)

                pltpu.make_async_copy(
                    g_hbm.at[srcv[q]], rows[b], gsems[b]).wait()
                pltpu.make_async_copy(dst2_hbm.at[0], dstv[q], dsems[q]).wait()
                pltpu.async_copy(rows[b], acc.at[dstv[q]], s2sems[b], add=True)

                @pl.when(i + KSC < NB_E)
                def _():
                    start_gather(i + KSC, (u + KSC) % NIB, (u + KSC) % NRB, off)
            return 0
        lax.fori_loop(0, NB_E // NIB, step, 0)

        for k in range(KSC):
            i = NB_E - KSC + k
            wait_scat(i % NRB, i % NIB)

        plsc.subcore_barrier()

        def wout(k, _):
            r0 = sid * RPT + k * ZR
            pltpu.async_copy(acc.at[pl.ds(r0, ZR)],
                             out_hbm.at[fc, pl.ds(r0, ZR)], wsem)
            return 0
        lax.fori_loop(0, RPT // ZR, wout, 0)

        def wdrain(k, _):
            r0 = sid * RPT
            pltpu.make_async_copy(acc.at[pl.ds(r0, ZR)],
                                  out_hbm.at[fc, pl.ds(r0, ZR)], wsem).wait()
            return 0
        lax.fori_loop(0, RPT // ZR, wdrain, 0)
        if j + 1 < NF // 2:
            plsc.subcore_barrier()
            zero_acc()
            plsc.subcore_barrier()


@functools.cache
def _sc_scat():
    return pl.kernel(
        _sc_scat_body,
        out_type=jax.ShapeDtypeStruct((NF, NP, FC), jnp.float32),
        mesh=_mesh(),
        scratch_types=[
            pltpu.VMEM_SHARED((NP, FC), jnp.float32),
            [pltpu.VMEM((BE,), jnp.int32) for _ in range(NIB)],
            [pltpu.VMEM((BE,), jnp.int32) for _ in range(NIB)],
            [pltpu.VMEM((BE, FC), jnp.float32) for _ in range(NRB)],
            [pltpu.SemaphoreType.DMA for _ in range(NRB)],
            [pltpu.SemaphoreType.DMA for _ in range(NIB)],
            [pltpu.SemaphoreType.DMA for _ in range(NIB)],
            [pltpu.SemaphoreType.DMA for _ in range(NRB)],
            pltpu.SemaphoreType.DMA,
        ],
    )
assert NB_E % NIB == 0 and NIB % NRB == 0


def _tc1_body(x_ref, w_ref, deg_ref, out_ref):
    dinv = lax.rsqrt(deg_ref[...] + 1.0)
    h = jnp.dot(x_ref[...], w_ref[...],
                preferred_element_type=jnp.float32)
    g = h * dinv
    for f in range(NF):
        out_ref[f] = g[:, f * FC:(f + 1) * FC]


def _tc1(xp, w1p, deg2):
    return pl.pallas_call(
        _tc1_body,
        grid=(NBLK,),
        in_specs=[
            pl.BlockSpec((BN, 16), lambda i: (i, 0)),
            pl.BlockSpec((16, DH), lambda i: (0, 0)),
            pl.BlockSpec((BN, 1), lambda i: (i, 0)),
        ],
        out_specs=pl.BlockSpec((NF, BN, FC), lambda i: (0, i, 0)),
        out_shape=jax.ShapeDtypeStruct((NF, NP, FC), jnp.float32),
    )(xp, w1p, deg2)


def _tc_mid_body(r_ref, g_ref, deg_ref, b_ref, w_ref, out_ref):
    dinv = lax.rsqrt(deg_ref[...] + 1.0)
    acc = jnp.zeros((BN, DH), jnp.float32)
    for f in range(NF):
        xf = jnp.maximum(
            (r_ref[f] + g_ref[f]) * dinv + b_ref[0:1, f * FC:(f + 1) * FC], 0.0)
        acc = acc + jnp.dot(xf, w_ref[f * FC:(f + 1) * FC, :],
                            preferred_element_type=jnp.float32)
    gn = acc * dinv
    for f in range(NF):
        out_ref[f] = gn[:, f * FC:(f + 1) * FC]


def _tc_mid(r, g, deg2, b, w):
    return pl.pallas_call(
        _tc_mid_body,
        grid=(NBLK,),
        in_specs=[
            pl.BlockSpec((NF, BN, FC), lambda i: (0, i, 0)),
            pl.BlockSpec((NF, BN, FC), lambda i: (0, i, 0)),
            pl.BlockSpec((BN, 1), lambda i: (i, 0)),
            pl.BlockSpec((1, DH), lambda i: (0, 0)),
            pl.BlockSpec((DH, DH), lambda i: (0, 0)),
        ],
        out_specs=pl.BlockSpec((NF, BN, FC), lambda i: (0, i, 0)),
        out_shape=jax.ShapeDtypeStruct((NF, NP, FC), jnp.float32),
    )(r, g, deg2, b, w)


def _tc4_body(r_ref, g_ref, deg_ref, b_ref, batch_ref, wl_ref, bl_ref,
              out_ref, sums_ref, cnt_ref):
    i = pl.program_id(0)

    @pl.when(i == 0)
    def _():
        sums_ref[...] = jnp.zeros((NG, DH), jnp.float32)
        cnt_ref[...] = jnp.zeros((NG, 1), jnp.float32)

    dinv = lax.rsqrt(deg_ref[...] + 1.0)
    bvec = batch_ref[...]
    gid = lax.broadcasted_iota(jnp.int32, (BN, NG), 1)
    P = jnp.where(bvec == gid, 1.0, 0.0)
    ones = jnp.ones((BN, 1), jnp.float32)
    cnt_ref[...] += lax.dot_general(
        P, ones, (((0,), (0,)), ((), ())),
        preferred_element_type=jnp.float32)
    for f in range(NF):
        h3f = (r_ref[f] + g_ref[f]) * dinv + b_ref[0:1, f * FC:(f + 1) * FC]
        sums_ref[:, f * FC:(f + 1) * FC] += lax.dot_general(
            P, h3f, (((0,), (0,)), ((), ())),
            preferred_element_type=jnp.float32)

    @pl.when(i == NBLK - 1)
    def _():
        pooled = sums_ref[...] / jnp.maximum(cnt_ref[...], 1.0)
        out_ref[...] = jnp.dot(pooled, wl_ref[...],
                               preferred_element_type=jnp.float32) + bl_ref[...]


def _tc4(r, g, deg2, b, batchp, wl, bl):
    return pl.pallas_call(
        _tc4_body,
        grid=(NBLK,),
        in_specs=[
            pl.BlockSpec((NF, BN, FC), lambda i: (0, i, 0)),
            pl.BlockSpec((NF, BN, FC), lambda i: (0, i, 0)),
            pl.BlockSpec((BN, 1), lambda i: (i, 0)),
            pl.BlockSpec((1, DH), lambda i: (0, 0)),
            pl.BlockSpec((BN, 1), lambda i: (i, 0)),
            pl.BlockSpec((DH, 1), lambda i: (0, 0)),
            pl.BlockSpec((1, 1), lambda i: (0, 0)),
        ],
        out_specs=pl.BlockSpec((NG, 1), lambda i: (0, 0)),
        out_shape=jax.ShapeDtypeStruct((NG, 1), jnp.float32),
        scratch_shapes=[
            pltpu.VMEM((NG, DH), jnp.float32),
            pltpu.VMEM((NG, 1), jnp.float32),
        ],
    )(r, g, deg2, b, batchp, wl, bl)


def kernel(x, edge_index, batch, W1, b1, W2, b2, W3, b3, Wlin, blin):
    xp = jnp.pad(x, ((0, NP - N), (0, 16 - x.shape[1])))
    w1p = jnp.pad(W1, ((0, 16 - W1.shape[0]), (0, 0)))
    padidx = N + (jnp.arange(EP - E, dtype=jnp.int32) % (NP - N))
    src = jnp.concatenate([edge_index[0], padidx]).reshape(EP // BE, BE)
    dst = jnp.concatenate([edge_index[1], padidx])
    dst2 = dst.reshape(EP // BE, BE)
    batchp = jnp.pad(batch, (0, NP - N), constant_values=NG).reshape(NP, 1)

    deg2 = _sc_deg()(dst2).sum(axis=0).reshape(NP, 1)
    g1 = _tc1(xp, w1p, deg2)
    r1 = _sc_scat()(g1.reshape(NF * NP, FC), src, dst2)
    g2 = _tc_mid(r1, g1, deg2, b1.reshape(1, DH), W2)
    r2 = _sc_scat()(g2.reshape(NF * NP, FC), src, dst2)
    g3 = _tc_mid(r2, g2, deg2, b2.reshape(1, DH), W3)
    r3 = _sc_scat()(g3.reshape(NF * NP, FC), src, dst2)
    return _tc4(r3, g3, deg2, b3.reshape(1, DH), batchp, Wlin,
                blin.reshape(1, 1))
